# baseline (speedup 1.0000x reference)
import functools

import jax
import jax.numpy as jnp
from jax import lax
from jax.experimental import pallas as pl
from jax.experimental.pallas import tpu as pltpu
from jax.experimental.pallas import tpu_sc as plsc

_P_MAX = 4.1
_P_MIN = 0.0
_N = 64
_M = 256
_S = 512
_NPIV = _N - 2

_L = 16
_NC, _NS = 2, 16
_NW = _NC * _NS
_K = 128
_MB = _K // _NW
_MT = _M - _K
_NPIV_PAD = 64

_mesh = plsc.VectorSubcoreMesh(core_axis_name="c", subcore_axis_name="s")


@functools.partial(
    pl.kernel,
    mesh=_mesh,
    compiler_params=pltpu.CompilerParams(needs_layout_passes=False),
    out_type=jax.ShapeDtypeStruct((_K, _S), jnp.float32),
    scratch_types=[
        pltpu.VMEM((_L,), jnp.float32),
        pltpu.VMEM((_NPIV_PAD,), jnp.float32),
        pltpu.VMEM((_MB, _S), jnp.float32),
        pltpu.VMEM((_MB, _S), jnp.float32),
        pltpu.SemaphoreType.DMA,
        pltpu.SemaphoreType.DMA,
        pltpu.SemaphoreType.DMA,
        pltpu.SemaphoreType.DMA,
        pltpu.SemaphoreType.DMA,
    ],
)
def _pw_kernel(lam_hbm, piv_hbm, alphas_hbm, out_hbm,
               lam_v, piv_v, left_v, right_v,
               sem_l, sem_r, sem_l1, sem_r1, sem_o):
    wid = lax.axis_index("s") * _NC + lax.axis_index("c")
    m0 = wid * _MB

    piv_v[pl.ds(_NPIV_PAD - _L, _L)] = jnp.full((_L,), jnp.float32(_P_MAX))
    copy_lam = pltpu.async_copy(lam_hbm, lam_v.at[pl.ds(0, 1)], sem_l)
    copy_piv = pltpu.async_copy(piv_hbm, piv_v.at[pl.ds(0, _NPIV)], sem_r)
    copy_lam.wait()
    copy_piv.wait()

    lam_s = lam_v[...][0]
    lam = jnp.full((_L,), lam_s)
    acc = jnp.zeros((_L,), jnp.int32)
    lmax = jnp.full((_L,), jnp.float32(_P_MIN))
    rmin = jnp.full((_L,), jnp.float32(_P_MAX))
    for j in range(_NPIV_PAD // _L):
        pv = piv_v[pl.ds(j * _L, _L)]
        le = pv <= lam
        acc = acc + jnp.where(le, jnp.int32(1), jnp.int32(0))
        lmax = jnp.maximum(lmax, jnp.where(le, pv, jnp.float32(_P_MIN)))
        rmin = jnp.minimum(rmin, jnp.where(le, jnp.float32(_P_MAX), pv))
    cnt = jnp.sum(acc)

    _H = _MB // 2
    copy_l0 = pltpu.async_copy(
        alphas_hbm.at[cnt, pl.ds(m0, _H), :], left_v.at[pl.ds(0, _H), :],
        sem_l)
    copy_r0 = pltpu.async_copy(
        alphas_hbm.at[cnt + 1, pl.ds(m0, _H), :], right_v.at[pl.ds(0, _H), :],
        sem_r)
    copy_l1 = pltpu.async_copy(
        alphas_hbm.at[cnt, pl.ds(m0 + _H, _H), :],
        left_v.at[pl.ds(_H, _H), :], sem_l1)
    copy_r1 = pltpu.async_copy(
        alphas_hbm.at[cnt + 1, pl.ds(m0 + _H, _H), :],
        right_v.at[pl.ds(_H, _H), :], sem_r1)

    lp = jnp.max(lmax)
    rp = jnp.min(rmin)
    kl = jnp.full((_L,), lam_s - lp) / jnp.full((_L,), rp - lp)
    kr = jnp.float32(1.0) - kl

    def blend_rows(r_lo, r_hi):
        def body(c, carry):
            off = c * _L
            for r in range(r_lo, r_hi):
                left_v[r, pl.ds(off, _L)] = (kl * left_v[r, pl.ds(off, _L)]
                                             + kr * right_v[r, pl.ds(off, _L)])
            return carry
        lax.fori_loop(0, _S // _L, body, jnp.int32(0))

    copy_l0.wait()
    copy_r0.wait()
    blend_rows(0, _H)
    out0 = pltpu.async_copy(
        left_v.at[pl.ds(0, _H), :], out_hbm.at[pl.ds(m0, _H), :], sem_o)
    copy_l1.wait()
    copy_r1.wait()
    blend_rows(_H, _MB)
    out0.wait()
    pltpu.sync_copy(left_v.at[pl.ds(_H, _H), :],
                    out_hbm.at[pl.ds(m0 + _H, _H), :])


def _tc_body(lam_ref, piv_ref, alphas_ref, out_ref, lbuf, rbuf, sem_l, sem_r):
    lam_s = lam_ref[0]
    pv = piv_ref[...]
    le = pv <= lam_s
    cnt = jnp.sum(le.astype(jnp.int32))
    copy_l = pltpu.make_async_copy(
        alphas_ref.at[cnt, pl.ds(_K, _MT), :], lbuf, sem_l)
    copy_r = pltpu.make_async_copy(
        alphas_ref.at[cnt + 1, pl.ds(_K, _MT), :], rbuf, sem_r)
    copy_l.start()
    copy_r.start()
    lp = jnp.maximum(jnp.float32(_P_MIN),
                     jnp.max(jnp.where(le, pv, jnp.float32(_P_MIN))))
    rp = jnp.minimum(jnp.float32(_P_MAX),
                     jnp.min(jnp.where(le, jnp.float32(_P_MAX), pv)))
    kl = (lam_s - lp) / (rp - lp)
    kr = jnp.float32(1.0) - kl
    copy_l.wait()
    copy_r.wait()
    out_ref[...] = kl * lbuf[...] + kr * rbuf[...]


_tc_blend = pl.pallas_call(
    _tc_body,
    out_shape=jax.ShapeDtypeStruct((_MT, _S), jnp.float32),
    in_specs=[
        pl.BlockSpec(memory_space=pltpu.SMEM),
        pl.BlockSpec(memory_space=pltpu.VMEM),
        pl.BlockSpec(memory_space=pl.ANY),
    ],
    out_specs=pl.BlockSpec(memory_space=pltpu.VMEM),
    scratch_shapes=[
        pltpu.VMEM((_MT, _S), jnp.float32),
        pltpu.VMEM((_MT, _S), jnp.float32),
        pltpu.SemaphoreType.DMA,
        pltpu.SemaphoreType.DMA,
    ],
)


def kernel(lam, alphas, pivots):
    lam1 = lam.reshape(1)
    sc_out = _pw_kernel(lam1, pivots, alphas)
    tc_out = _tc_blend(lam1, pivots, alphas)
    return jnp.concatenate([sc_out, tc_out], axis=0)


# speedup vs baseline: 1.0908x; 1.0908x over previous
import functools

import jax
import jax.numpy as jnp
from jax import lax
from jax.experimental import pallas as pl
from jax.experimental.pallas import tpu as pltpu
from jax.experimental.pallas import tpu_sc as plsc

_P_MAX = 4.1
_P_MIN = 0.0
_N = 64
_M = 256
_S = 512
_NPIV = _N - 2

_L = 16
_NC, _NS = 2, 16
_NW = _NC * _NS
_MB = _M // _NW
_NPIV_PAD = 64

_mesh = plsc.VectorSubcoreMesh(core_axis_name="c", subcore_axis_name="s")


@functools.partial(
    pl.kernel,
    mesh=_mesh,
    compiler_params=pltpu.CompilerParams(needs_layout_passes=False),
    out_type=jax.ShapeDtypeStruct((_M, _S), jnp.float32),
    scratch_types=[
        pltpu.VMEM((_L,), jnp.float32),
        pltpu.VMEM((_NPIV_PAD,), jnp.float32),
        pltpu.VMEM((_MB, _S), jnp.float32),
        pltpu.VMEM((_MB, _S), jnp.float32),
        pltpu.SemaphoreType.DMA,
        pltpu.SemaphoreType.DMA,
    ],
)
def _pw_kernel(lam_hbm, piv_hbm, alphas_hbm, out_hbm,
               lam_v, piv_v, left_v, right_v, sem_l, sem_r):
    wid = lax.axis_index("s") * _NC + lax.axis_index("c")
    m0 = wid * _MB

    piv_v[pl.ds(_NPIV_PAD - _L, _L)] = jnp.full((_L,), jnp.float32(_P_MAX))
    copy_lam = pltpu.async_copy(lam_hbm, lam_v.at[pl.ds(0, 1)], sem_l)
    copy_piv = pltpu.async_copy(piv_hbm, piv_v.at[pl.ds(0, _NPIV)], sem_r)
    copy_lam.wait()
    copy_piv.wait()

    lam_s = lam_v[...][0]
    lam = jnp.full((_L,), lam_s)
    acc = jnp.zeros((_L,), jnp.int32)
    lmax = jnp.full((_L,), jnp.float32(_P_MIN))
    rmin = jnp.full((_L,), jnp.float32(_P_MAX))
    for j in range(_NPIV_PAD // _L):
        pv = piv_v[pl.ds(j * _L, _L)]
        le = pv <= lam
        acc = acc + jnp.where(le, jnp.int32(1), jnp.int32(0))
        lmax = jnp.maximum(lmax, jnp.where(le, pv, jnp.float32(_P_MIN)))
        rmin = jnp.minimum(rmin, jnp.where(le, jnp.float32(_P_MAX), pv))
    cnt = jnp.sum(acc)

    copy_l = pltpu.async_copy(
        alphas_hbm.at[cnt, pl.ds(m0, _MB), :], left_v, sem_l)
    copy_r = pltpu.async_copy(
        alphas_hbm.at[cnt + 1, pl.ds(m0, _MB), :], right_v, sem_r)

    lp = jnp.max(lmax)
    rp = jnp.min(rmin)
    kl = jnp.full((_L,), lam_s - lp) / jnp.full((_L,), rp - lp)
    kr = jnp.float32(1.0) - kl

    copy_l.wait()
    copy_r.wait()

    @plsc.parallel_loop(0, _S // _L, 1, unroll=4)
    def _(c):
        off = c * _L
        for r in range(_MB):
            left_v[r, pl.ds(off, _L)] = (kl * left_v[r, pl.ds(off, _L)]
                                         + kr * right_v[r, pl.ds(off, _L)])

    pltpu.sync_copy(left_v, out_hbm.at[pl.ds(m0, _MB), :])


def kernel(lam, alphas, pivots):
    return _pw_kernel(lam.reshape(1), pivots, alphas)
